# baseline (speedup 1.0000x reference)
import functools

import jax
import jax.numpy as jnp
from jax import lax
from jax.experimental import pallas as pl

_B = 16384
_E = 64
_K = 1024
_L = 3
_BETA = 0.25


def _dot(a, b):
    return jnp.dot(a.astype(jnp.bfloat16), b.astype(jnp.bfloat16),
                   preferred_element_type=jnp.float32)


def _dot_exact(a, b, dims):
    return lax.dot_general(a, b, (dims, ((), ())),
                           preferred_element_type=jnp.float32,
                           precision=lax.Precision.HIGHEST)


def _fused_kernel(x_ref,
                  w1, b1, w2, b2, w3, b3, w4, b4,
                  cb_ref, cb2_ref,
                  v1, c1, v2, c2, v3, c3, v4, c4,
                  out_ref, xq_ref, idx_ref, loss_ref):
    i = pl.program_id(0)
    h = x_ref[:]
    h = jnp.maximum(_dot(h, w1[:]) + b1[:], 0.0)
    h = jnp.maximum(_dot(h, w2[:]) + b2[:], 0.0)
    h = jnp.maximum(_dot(h, w3[:]) + b3[:], 0.0)
    e = _dot(h, w4[:]) + b4[:]

    bm = e.shape[0]
    lane_iota = lax.broadcasted_iota(jnp.int32, (bm, _K), 1)

    r = e
    xq = jnp.zeros_like(e)
    s_parts = []
    idx_cols = []
    for l in range(_L):
        cb = cb_ref[l]
        r2 = jnp.sum(r * r, axis=1, keepdims=True)
        rc = lax.dot_general(r.astype(jnp.bfloat16), cb.astype(jnp.bfloat16),
                             (((1,), (1,)), ((), ())),
                             preferred_element_type=jnp.float32)
        cb2 = cb2_ref[l]
        d = (r2 - 2.0 * rc) + cb2
        dmin = jnp.min(d, axis=1, keepdims=True)
        idx = jnp.min(jnp.where(d <= dmin, lane_iota, _K), axis=1,
                      keepdims=True)
        onehot = (lane_iota == idx).astype(jnp.float32)
        q = _dot_exact(onehot, cb, ((1,), (0,)))
        diff = r - q
        rowsq = jnp.sum(diff * diff, axis=1, keepdims=True)
        s_parts.append(jnp.sum(rowsq, axis=0, keepdims=True))
        idx_cols.append(idx)
        xq = xq + q
        r = diff

    idx_ref[:] = jnp.concatenate(idx_cols, axis=1)
    xq_ref[:] = xq

    g = jnp.maximum(_dot(xq, v1[:]) + c1[:], 0.0)
    g = jnp.maximum(_dot(g, v2[:]) + c2[:], 0.0)
    g = jnp.maximum(_dot(g, v3[:]) + c3[:], 0.0)
    out_ref[:] = _dot(g, v4[:]) + c4[:]

    s = jnp.concatenate(s_parts, axis=1)

    @pl.when(i == 0)
    def _():
        loss_ref[:] = s

    @pl.when(i > 0)
    def _():
        loss_ref[:] = loss_ref[:] + s


@functools.partial(jax.jit, static_argnames=("bm",))
def _modality(x, enc, cbs, dec, bm=1024):
    b, in_dim = x.shape
    grid = b // bm
    (w1, b1), (w2, b2), (w3, b3), (w4, b4) = enc
    (v1, c1), (v2, c2), (v3, c3), (v4, c4) = dec
    row = lambda v: v.reshape(1, -1)
    const = lambda a: pl.BlockSpec(a.shape, lambda i: (0,) * a.ndim)

    cb2 = jnp.sum(cbs ** 2, axis=2)[:, None, :]
    args = [x,
            w1, row(b1), w2, row(b2), w3, row(b3), w4, row(b4),
            cbs, cb2,
            v1, row(c1), v2, row(c2), v3, row(c3), v4, row(c4)]
    in_specs = [pl.BlockSpec((bm, in_dim), lambda i: (i, 0))]
    in_specs += [const(a) for a in args[1:]]

    out_shape = [
        jax.ShapeDtypeStruct((b, in_dim), jnp.float32),
        jax.ShapeDtypeStruct((b, _E), jnp.float32),
        jax.ShapeDtypeStruct((b, _L), jnp.int32),
        jax.ShapeDtypeStruct((1, _L), jnp.float32),
    ]
    out_specs = [
        pl.BlockSpec((bm, in_dim), lambda i: (i, 0)),
        pl.BlockSpec((bm, _E), lambda i: (i, 0)),
        pl.BlockSpec((bm, _L), lambda i: (i, 0)),
        pl.BlockSpec((1, _L), lambda i: (0, 0)),
    ]
    out, xq, idx, s = pl.pallas_call(
        _fused_kernel,
        grid=(grid,),
        in_specs=in_specs,
        out_specs=out_specs,
        out_shape=out_shape,
    )(*args)
    m = s[0] / b
    rq_loss = jnp.mean(m + _BETA * m)
    return out, xq, idx, rq_loss


def kernel(x, y, z, params, labels, labels_2, labels_3):
    out, x_q, indices, rq_loss = _modality(
        x, params['encoder'], params['codebooks'], params['decoder'])
    pic_out, y_q, indices_2, rq_loss_2 = _modality(
        y, params['pic_encoder'], params['pic_codebooks'], params['pic_decoder'])
    text_out, z_q, indices_3, rq_loss_3 = _modality(
        z, params['text_encoder'], params['text_codebooks'], params['text_decoder'])
    return (out, pic_out, text_out, rq_loss, rq_loss_2, rq_loss_3,
            indices, indices_2, indices_3, x_q, y_q, z_q)
